# baseline (speedup 1.0000x reference)
import functools
import math

import jax
import jax.numpy as jnp
from jax import lax
from jax.experimental import pallas as pl
from jax.experimental.pallas import tpu as pltpu
from jax.experimental.pallas import tpu_sc as plsc

_CHUNK = 128
_SH = 2
_TBLK = 512
_VMEM = 56 * 1024 * 1024


def _sc_gather(table, ei_arr, oi_arr):
    NW, n_chunks, _ = ei_arr.shape
    D = table.shape[1]
    q_rows = n_chunks * _CHUNK
    half = _SH
    sc_q = half * _CHUNK
    n_super = n_chunks // half
    assert n_chunks % half == 0 and n_super % 2 == 0
    mesh = plsc.VectorSubcoreMesh(core_axis_name="c", subcore_axis_name="s")

    @functools.partial(
        pl.kernel,
        out_type=jax.ShapeDtypeStruct((NW * q_rows, 2 * D), jnp.float32),
        mesh=mesh,
        scratch_types=[
            pltpu.VMEM((2 * n_chunks, _CHUNK), jnp.int32),
            pltpu.VMEM((2, 2 * sc_q, D), jnp.float32),
            pltpu.SemaphoreType.DMA,
            pltpu.SemaphoreType.DMA,
            pltpu.SemaphoreType.DMA,
            pltpu.SemaphoreType.DMA,
        ],
        compiler_params=pltpu.CompilerParams(use_tc_tiling_on_sc=False),
    )
    def k(tab, ei, oi, out_hbm, idx_v, rows2, g0, g1, o0, o1):
        wid = lax.axis_index("s") * 2 + lax.axis_index("c")
        base = wid * q_rows
        gsems = (g0, g1)
        osems = (o0, o1)
        pltpu.sync_copy(ei.at[wid], idx_v.at[pl.ds(0, n_chunks)])
        pltpu.sync_copy(oi.at[wid], idx_v.at[pl.ds(n_chunks, n_chunks)])

        def drain_write(buf, g, sem):
            q0 = base + g * sc_q
            pltpu.make_async_copy(
                buf.at[pl.ds(0, sc_q)],
                out_hbm.at[pl.ds(q0, sc_q), pl.ds(0, D)],
                sem,
            ).wait()
            pltpu.make_async_copy(
                buf.at[pl.ds(sc_q, sc_q)],
                out_hbm.at[pl.ds(q0, sc_q), pl.ds(D, D)],
                sem,
            ).wait()

        def fire_gathers(buf, g, sem):
            for par in range(2):
                for sblk in range(half):
                    j = par * n_chunks + g * half + sblk
                    pltpu.async_copy(
                        tab.at[idx_v.at[j]],
                        buf.at[pl.ds((par * half + sblk) * _CHUNK, _CHUNK)],
                        sem,
                    )

        def drain_gathers(buf, g, sem):
            for par in range(2):
                for sblk in range(half):
                    j = par * n_chunks + g * half + sblk
                    pltpu.make_async_copy(
                        tab.at[idx_v.at[j]],
                        buf.at[pl.ds((par * half + sblk) * _CHUNK, _CHUNK)],
                        sem,
                    ).wait()

        def fire_write(buf, g, sem):
            q0 = base + g * sc_q
            pltpu.async_copy(
                buf.at[pl.ds(0, sc_q)],
                out_hbm.at[pl.ds(q0, sc_q), pl.ds(0, D)],
                sem,
            )
            pltpu.async_copy(
                buf.at[pl.ds(sc_q, sc_q)],
                out_hbm.at[pl.ds(q0, sc_q), pl.ds(D, D)],
                sem,
            )

        fire_gathers(rows2.at[0], 0, gsems[0])

        @pl.loop(0, n_super, step=2)
        def _(gg):
            for cur in range(2):
                g = gg + cur
                nxt = 1 - cur

                @pl.when(g + 1 < n_super)
                def _():
                    @pl.when(g >= 1)
                    def _():
                        drain_write(rows2.at[nxt], g - 1, osems[nxt])

                    fire_gathers(rows2.at[nxt], g + 1, gsems[nxt])

                drain_gathers(rows2.at[cur], g, gsems[cur])
                fire_write(rows2.at[cur], g, osems[cur])

        for cur in range(2):
            drain_write(rows2.at[cur], n_super - 2 + cur, osems[cur])

    return k(table, ei_arr, oi_arr)


def _gate_x_body(gx_ref, pe_ref, w_ref, b_ref, *rest, d, nt, scale):
    out_ref, ex_ref = rest[-2], rest[-1]
    w = w_ref[...]
    wt = jnp.transpose(w)
    pe_col = jnp.transpose(pe_ref[0])
    bcat = jnp.concatenate([b_ref[...], b_ref[...]], axis=0)

    @pl.loop(0, nt)
    def _(t):
        slab = gx_ref[0, pl.ds(t * _TBLK, _TBLK), :]
        sx = jnp.transpose(slab) * scale
        ex_ref[:, 0, :, pl.ds(t * _TBLK, _TBLK)] = sx.reshape(2, d, _TBLK)
        xx = sx + pe_col
        r_top = jnp.dot(wt, xx[:d], preferred_element_type=jnp.float32)
        r_bot = jnp.dot(wt, xx[d:], preferred_element_type=jnp.float32)
        r = jnp.concatenate([r_top, r_bot], axis=0) + bcat + 1.0
        out_ref[:, 0, :, pl.ds(t * _TBLK, _TBLK)] = (xx * r).reshape(2, d, _TBLK)


def _scale_t_body(gm_ref, em_ref, *, d, nt, scale):
    @pl.loop(0, nt)
    def _(t):
        slab = gm_ref[0, pl.ds(t * _TBLK, _TBLK), :]
        sx = jnp.transpose(slab) * scale
        em_ref[:, 0, :, pl.ds(t * _TBLK, _TBLK)] = sx.reshape(2, d, _TBLK)


def _gate_x(gx3, peP, W, bcol, K, k0, bufs=None):
    Kh, B, D2 = gx3.shape
    d = D2 // 2
    in_specs = [
        pl.BlockSpec((1, B, D2), lambda k: (k, 0, 0)),
        pl.BlockSpec((1, 1, D2), lambda k: (k0 + k, 0, 0)),
        pl.BlockSpec((d, d), lambda k: (0, 0)),
        pl.BlockSpec((d, 1), lambda k: (0, 0)),
    ]
    operands = [gx3, peP, W, bcol]
    aliases = {}
    if bufs is not None:
        in_specs += [pl.BlockSpec(memory_space=pltpu.MemorySpace.HBM)] * 2
        operands += list(bufs)
        aliases = {4: 0, 5: 1}
    return pl.pallas_call(
        functools.partial(_gate_x_body, d=d, nt=B // _TBLK, scale=math.sqrt(d)),
        grid=(Kh,),
        in_specs=in_specs,
        out_specs=[pl.BlockSpec((2, 1, d, B), lambda k: (0, k0 + k, 0, 0))] * 2,
        out_shape=[jax.ShapeDtypeStruct((2, K, d, B), jnp.float32)] * 2,
        input_output_aliases=aliases,
        compiler_params=pltpu.CompilerParams(vmem_limit_bytes=_VMEM),
    )(*operands)


def _scale_t(gm3):
    K, B, D2 = gm3.shape
    d = D2 // 2
    return pl.pallas_call(
        functools.partial(_scale_t_body, d=d, nt=B // _TBLK, scale=math.sqrt(d)),
        grid=(K,),
        in_specs=[pl.BlockSpec((1, B, D2), lambda k: (k, 0, 0))],
        out_specs=pl.BlockSpec((2, 1, d, B), lambda k: (0, k, 0, 0)),
        out_shape=jax.ShapeDtypeStruct((2, K, d, B), jnp.float32),
        compiler_params=pltpu.CompilerParams(vmem_limit_bytes=_VMEM),
    )(gm3)


def kernel(x, m, x_table, m_table, W, b, pe):
    B, L = x.shape
    V, D = x_table.shape
    NW = 32
    K = L // 2

    xT = jnp.transpose(x)
    mT = jnp.transpose(m)
    K2 = 48
    nc_a = K2 * B // NW // _CHUNK
    nc_b = (K - K2) * B // NW // _CHUNK
    n_chunks = K * B // NW // _CHUNK
    xe_a = xT[:K2].reshape(NW, nc_a, _CHUNK)
    xo_a = xT[K:K + K2].reshape(NW, nc_a, _CHUNK)
    xe_b = xT[K2:K].reshape(NW, nc_b, _CHUNK)
    xo_b = xT[K + K2:].reshape(NW, nc_b, _CHUNK)
    me = mT[:K].reshape(NW, n_chunks, _CHUNK)
    mo = mT[K:].reshape(NW, n_chunks, _CHUNK)

    gx_a = _sc_gather(x_table, xe_a, xo_a)
    tie_a = (gx_a[0, 0] * 0.0).astype(jnp.int32)
    gx_b = _sc_gather(x_table, xe_b + tie_a, xo_b + tie_a)
    tie_b = (gx_b[0, 0] * 0.0).astype(jnp.int32)
    gm = _sc_gather(m_table, me + tie_b, mo + tie_b)
    gx3_a = gx_a.reshape(K2, B, 2 * D)
    gx3_b = gx_b.reshape(K - K2, B, 2 * D)
    gm3 = gm.reshape(K, B, 2 * D)
    pe0 = pe[0, :L, :]
    peP = jnp.concatenate([pe0[:K], pe0[K:]], axis=1)[:, None, :]

    oa, ea = _gate_x(gx3_a, peP, W, b[:, None], K, 0)
    out4, ex4 = _gate_x(gx3_b, peP, W, b[:, None], K, K2, bufs=(oa, ea))
    em4 = _scale_t(gm3)
    tr = lambda a: jnp.transpose(a.reshape(L, D, B), (2, 0, 1))
    return tr(out4), tr(ex4), tr(em4)


# speedup vs baseline: 1.0410x; 1.0410x over previous
import functools
import math

import jax
import jax.numpy as jnp
from jax import lax
from jax.experimental import pallas as pl
from jax.experimental.pallas import tpu as pltpu
from jax.experimental.pallas import tpu_sc as plsc

_CHUNK = 128
_SH = 2
_TBLK = 512
_VMEM = 56 * 1024 * 1024


def _sc_gather(table, ei_arr, oi_arr):
    NW, n_chunks, _ = ei_arr.shape
    D = table.shape[1]
    q_rows = n_chunks * _CHUNK
    half = _SH
    sc_q = half * _CHUNK
    n_super = n_chunks // half
    assert n_chunks % half == 0 and n_super % 2 == 0
    mesh = plsc.VectorSubcoreMesh(core_axis_name="c", subcore_axis_name="s")

    @functools.partial(
        pl.kernel,
        out_type=jax.ShapeDtypeStruct((NW * q_rows, 2 * D), jnp.float32),
        mesh=mesh,
        scratch_types=[
            pltpu.VMEM((2 * n_chunks, _CHUNK), jnp.int32),
            pltpu.VMEM((2, 2 * sc_q, D), jnp.float32),
            pltpu.SemaphoreType.DMA,
            pltpu.SemaphoreType.DMA,
            pltpu.SemaphoreType.DMA,
            pltpu.SemaphoreType.DMA,
        ],
        compiler_params=pltpu.CompilerParams(use_tc_tiling_on_sc=False),
    )
    def k(tab, ei, oi, out_hbm, idx_v, rows2, g0, g1, o0, o1):
        wid = lax.axis_index("s") * 2 + lax.axis_index("c")
        base = wid * q_rows
        gsems = (g0, g1)
        osems = (o0, o1)
        pltpu.sync_copy(ei.at[wid], idx_v.at[pl.ds(0, n_chunks)])
        pltpu.sync_copy(oi.at[wid], idx_v.at[pl.ds(n_chunks, n_chunks)])

        def drain_write(buf, g, sem):
            q0 = base + g * sc_q
            pltpu.make_async_copy(
                buf.at[pl.ds(0, sc_q)],
                out_hbm.at[pl.ds(q0, sc_q), pl.ds(0, D)],
                sem,
            ).wait()
            pltpu.make_async_copy(
                buf.at[pl.ds(sc_q, sc_q)],
                out_hbm.at[pl.ds(q0, sc_q), pl.ds(D, D)],
                sem,
            ).wait()

        def fire_gathers(buf, g, sem):
            for par in range(2):
                for sblk in range(half):
                    j = par * n_chunks + g * half + sblk
                    pltpu.async_copy(
                        tab.at[idx_v.at[j]],
                        buf.at[pl.ds((par * half + sblk) * _CHUNK, _CHUNK)],
                        sem,
                    )

        def drain_gathers(buf, g, sem):
            for par in range(2):
                for sblk in range(half):
                    j = par * n_chunks + g * half + sblk
                    pltpu.make_async_copy(
                        tab.at[idx_v.at[j]],
                        buf.at[pl.ds((par * half + sblk) * _CHUNK, _CHUNK)],
                        sem,
                    ).wait()

        def fire_write(buf, g, sem):
            q0 = base + g * sc_q
            pltpu.async_copy(
                buf.at[pl.ds(0, sc_q)],
                out_hbm.at[pl.ds(q0, sc_q), pl.ds(0, D)],
                sem,
            )
            pltpu.async_copy(
                buf.at[pl.ds(sc_q, sc_q)],
                out_hbm.at[pl.ds(q0, sc_q), pl.ds(D, D)],
                sem,
            )

        fire_gathers(rows2.at[0], 0, gsems[0])

        @pl.loop(0, n_super, step=2)
        def _(gg):
            for cur in range(2):
                g = gg + cur
                nxt = 1 - cur

                @pl.when(g + 1 < n_super)
                def _():
                    @pl.when(g >= 1)
                    def _():
                        drain_write(rows2.at[nxt], g - 1, osems[nxt])

                    fire_gathers(rows2.at[nxt], g + 1, gsems[nxt])

                drain_gathers(rows2.at[cur], g, gsems[cur])
                fire_write(rows2.at[cur], g, osems[cur])

        for cur in range(2):
            drain_write(rows2.at[cur], n_super - 2 + cur, osems[cur])

    return k(table, ei_arr, oi_arr)


def _gate_x_body(gx_ref, pe_ref, w_ref, b_ref, *rest, d, nt, scale):
    out_ref, ex_ref = rest[-2], rest[-1]
    w = w_ref[...]
    wt = jnp.transpose(w)
    pe_col = jnp.transpose(pe_ref[0])
    bcat = jnp.concatenate([b_ref[...], b_ref[...]], axis=0)

    @pl.loop(0, nt)
    def _(t):
        slab = gx_ref[pl.ds(t * _TBLK, _TBLK), :]
        sx = jnp.transpose(slab) * scale
        ex_ref[:, 0, :, pl.ds(t * _TBLK, _TBLK)] = sx.reshape(2, d, _TBLK)
        xx = sx + pe_col
        r_top = jnp.dot(wt, xx[:d], preferred_element_type=jnp.float32)
        r_bot = jnp.dot(wt, xx[d:], preferred_element_type=jnp.float32)
        r = jnp.concatenate([r_top, r_bot], axis=0) + bcat + 1.0
        out_ref[:, 0, :, pl.ds(t * _TBLK, _TBLK)] = (xx * r).reshape(2, d, _TBLK)


def _scale_t_body(gm_ref, em_ref, *, d, nt, scale):
    @pl.loop(0, nt)
    def _(t):
        slab = gm_ref[pl.ds(t * _TBLK, _TBLK), :]
        sx = jnp.transpose(slab) * scale
        em_ref[:, 0, :, pl.ds(t * _TBLK, _TBLK)] = sx.reshape(2, d, _TBLK)


def _gate_x(gx2, peP, W, bcol, B, K, k0, bufs=None):
    _, D2 = gx2.shape
    Kh = gx2.shape[0] // B
    d = D2 // 2
    in_specs = [
        pl.BlockSpec((B, D2), lambda k: (k, 0)),
        pl.BlockSpec((1, 1, D2), lambda k: (k0 + k, 0, 0)),
        pl.BlockSpec((d, d), lambda k: (0, 0)),
        pl.BlockSpec((d, 1), lambda k: (0, 0)),
    ]
    operands = [gx2, peP, W, bcol]
    aliases = {}
    if bufs is not None:
        in_specs += [pl.BlockSpec(memory_space=pltpu.MemorySpace.HBM)] * 2
        operands += list(bufs)
        aliases = {4: 0, 5: 1}
    return pl.pallas_call(
        functools.partial(_gate_x_body, d=d, nt=B // _TBLK, scale=math.sqrt(d)),
        grid=(Kh,),
        in_specs=in_specs,
        out_specs=[pl.BlockSpec((2, 1, d, B), lambda k: (0, k0 + k, 0, 0))] * 2,
        out_shape=[jax.ShapeDtypeStruct((2, K, d, B), jnp.float32)] * 2,
        input_output_aliases=aliases,
        compiler_params=pltpu.CompilerParams(vmem_limit_bytes=_VMEM),
    )(*operands)


def _scale_t(gm2, B):
    _, D2 = gm2.shape
    K = gm2.shape[0] // B
    d = D2 // 2
    return pl.pallas_call(
        functools.partial(_scale_t_body, d=d, nt=B // _TBLK, scale=math.sqrt(d)),
        grid=(K,),
        in_specs=[pl.BlockSpec((B, D2), lambda k: (k, 0))],
        out_specs=pl.BlockSpec((2, 1, d, B), lambda k: (0, k, 0, 0)),
        out_shape=jax.ShapeDtypeStruct((2, K, d, B), jnp.float32),
        compiler_params=pltpu.CompilerParams(vmem_limit_bytes=_VMEM),
    )(gm2)


def kernel(x, m, x_table, m_table, W, b, pe):
    B, L = x.shape
    V, D = x_table.shape
    NW = 32
    K = L // 2

    xT = jnp.transpose(x)
    mT = jnp.transpose(m)
    K2 = 48
    nc_a = K2 * B // NW // _CHUNK
    nc_b = (K - K2) * B // NW // _CHUNK
    n_chunks = K * B // NW // _CHUNK
    xe_a = xT[:K2].reshape(NW, nc_a, _CHUNK)
    xo_a = xT[K:K + K2].reshape(NW, nc_a, _CHUNK)
    xe_b = xT[K2:K].reshape(NW, nc_b, _CHUNK)
    xo_b = xT[K + K2:].reshape(NW, nc_b, _CHUNK)
    me = mT[:K].reshape(NW, n_chunks, _CHUNK)
    mo = mT[K:].reshape(NW, n_chunks, _CHUNK)

    gx_a = _sc_gather(x_table, xe_a, xo_a)
    tie_a = (gx_a[0, 0] * 0.0).astype(jnp.int32)
    gx_b = _sc_gather(x_table, xe_b + tie_a, xo_b + tie_a)
    gm = _sc_gather(m_table, me + tie_a, mo + tie_a)
    pe0 = pe[0, :L, :]
    peP = jnp.concatenate([pe0[:K], pe0[K:]], axis=1)[:, None, :]

    oa, ea = _gate_x(gx_a, peP, W, b[:, None], B, K, 0)
    out4, ex4 = _gate_x(gx_b, peP, W, b[:, None], B, K, K2, bufs=(oa, ea))
    em4 = _scale_t(gm, B)
    tr = lambda a: jnp.transpose(a.reshape(L, D, B), (2, 0, 1))
    return tr(out4), tr(ex4), tr(em4)
